# Initial kernel scaffold; baseline (speedup 1.0000x reference)
import jax
import jax.numpy as jnp
from jax.experimental import pallas as pl


def kernel(id1, W):
    raise NotImplementedError("write your pallas kernel here")


# speedup vs baseline: 27.1682x; 27.1682x over previous
import jax
import jax.numpy as jnp
from jax.experimental import pallas as pl


def _body(ids_ref, r_ref, w0_ref, w1_ref, out_ref):
    idf = ids_ref[...].astype(jnp.bfloat16)
    rep = jax.lax.dot_general(
        idf, r_ref[...],
        dimension_numbers=(((1,), (0,)), ((), ())),
        preferred_element_type=jnp.float32,
    )
    out_ref[...] = jnp.where(rep > 0.5, w1_ref[...], w0_ref[...])


def kernel(id1, W):
    N, J = id1.shape
    D = W.shape[1]
    JD = J * D
    B = 1024
    R = (jnp.arange(JD, dtype=jnp.int32)[None, :] // D
         == jnp.arange(J, dtype=jnp.int32)[:, None]).astype(jnp.bfloat16)
    w0t = jnp.tile(W[0], J)[None, :]
    w1t = jnp.tile(W[1], J)[None, :]
    out = pl.pallas_call(
        _body,
        grid=(N // B,),
        in_specs=[
            pl.BlockSpec((B, J), lambda i: (i, 0)),
            pl.BlockSpec((J, JD), lambda i: (0, 0)),
            pl.BlockSpec((1, JD), lambda i: (0, 0)),
            pl.BlockSpec((1, JD), lambda i: (0, 0)),
        ],
        out_specs=pl.BlockSpec((B, JD), lambda i: (i, 0)),
        out_shape=jax.ShapeDtypeStruct((N, JD), jnp.float32),
    )(id1, R, w0t, w1t)
    return out.reshape(N, J, D)
